# baseline (speedup 1.0000x reference)
import functools

import jax
import jax.numpy as jnp
from jax import lax
from jax.experimental import pallas as pl
from jax.experimental.pallas import tpu as pltpu
from jax.experimental.pallas import tpu_sc as plsc

N_NODES = 10000
N_EDGES = 320000
D = 128
BS = 1000

NW = 32
RPW = 320
NPAD = NW * RPW
ECAP = N_EDGES + 8192
CH = 2560
NV = CH // 16
NCH = N_EDGES // CH
NBLK = CH // 128
GGRP = 128


def _mm1_body(x_ref, wfs_ref, bfs_ref, wgn_ref, bgn_ref, t_ref, base_ref):
    x = x_ref[...]
    t_ref[...] = jnp.maximum(
        lax.dot_general(x, wfs_ref[...], (((1,), (1,)), ((), ()))) + bfs_ref[...], 0.0
    )
    base_ref[...] = lax.dot_general(x, wgn_ref[...], (((1,), (1,)), ((), ()))) + bgn_ref[...]


def _mm2_body(seg_ref, wgin_ref, bgin_ref, base_ref, out_ref):
    out_ref[...] = (
        lax.dot_general(seg_ref[...], wgin_ref[...], (((1,), (1,)), ((), ())))
        + bgin_ref[...]
        + base_ref[...]
    )


def _segmax_body(t_hbm, snd_hbm, rcv_hbm, out_hbm, wls_hbm, wlr_hbm,
                 acc, rv_buf, sv_buf, posbuf, padsrc, sbuf, rbuf, rows, sem):
    wid = lax.axis_index("s") * 2 + lax.axis_index("c")
    base = wid * RPW
    wbase = wid * ECAP
    trash_pos = wbase + ECAP - 8
    lane = lax.iota(jnp.int32, 16)
    dn = lax.GatherDimensionNumbers(offset_dims=(), collapsed_slice_dims=(0,),
                                    start_index_map=(0,))

    def dg(x, idx):
        return lax.gather(x, idx[:, None], dn, (1,),
                          mode=lax.GatherScatterMode.PROMISE_IN_BOUNDS)

    zero16f = jnp.zeros((16,), jnp.float32)
    zero16i = jnp.zeros((16,), jnp.int32)

    def zrow(i, _):
        for f in range(8):
            acc[i, pl.ds(f * 16, 16)] = zero16f
        return 0

    lax.fori_loop(0, RPW + 8, zrow, 0)

    def zpad(i, _):
        o = pl.multiple_of(i * 16, 16)
        padsrc[0, pl.ds(o, 16)] = zero16i
        padsrc[1, pl.ds(o, 16)] = zero16i + (base + RPW)
        return 0

    lax.fori_loop(0, 8, zpad, 0)

    def chunk_body(c, gcnt):
        off = pl.multiple_of(c * CH, 8)
        pltpu.sync_copy(rcv_hbm.at[pl.ds(off, CH)], rv_buf)
        pltpu.sync_copy(snd_hbm.at[pl.ds(off, CH)], sv_buf)

        def scan_body(i, gp):
            o = pl.multiple_of(i * 16, 16)
            rv = rv_buf[pl.ds(o, 16)]
            lowerm = jnp.minimum(jnp.maximum(rv - (base - 1), 0), 1)
            upperm = jnp.minimum(jnp.maximum((base + RPW) - rv, 0), 1)
            mi = lowerm * upperm
            x = mi
            for j in (1, 2, 4, 8):
                x = x + jnp.where(lane >= j, dg(x, jnp.maximum(lane - j, 0)), 0)
            a = x + (wbase + gp - 1)
            posbuf[pl.ds(o, 16)] = mi * (a - trash_pos) + trash_pos
            return gp + x[15]

        gp2 = lax.fori_loop(0, NV, scan_body, gcnt)

        cps = []
        for k in range(NBLK):
            ko = k * 128
            cps.append(pltpu.async_copy(
                sv_buf.at[pl.ds(ko, 128)], wls_hbm.at[posbuf.at[pl.ds(ko, 128)]], sem))
            cps.append(pltpu.async_copy(
                rv_buf.at[pl.ds(ko, 128)], wlr_hbm.at[posbuf.at[pl.ds(ko, 128)]], sem))
        for cp in cps:
            cp.wait()
        return gp2

    gcnt = lax.fori_loop(0, NCH, chunk_body, jnp.int32(0))

    def padpos(i, _):
        o = pl.multiple_of(i * 16, 16)
        posbuf[pl.ds(o, 16)] = lane + (wbase + gcnt + o)
        return 0

    lax.fori_loop(0, 8, padpos, 0)
    pltpu.async_copy(padsrc.at[0, pl.ds(0, 128)],
                     wls_hbm.at[posbuf.at[pl.ds(0, 128)]], sem).wait()
    pltpu.async_copy(padsrc.at[1, pl.ds(0, 128)],
                     wlr_hbm.at[posbuf.at[pl.ds(0, 128)]], sem).wait()

    ngrp = (gcnt + (GGRP - 1)) // GGRP

    def grp_body(g, _):
        goff = pl.multiple_of(g * GGRP, GGRP)
        pltpu.sync_copy(wls_hbm.at[pl.ds(pl.multiple_of(wbase + goff, 8), GGRP)], sbuf)
        pltpu.sync_copy(wlr_hbm.at[pl.ds(pl.multiple_of(wbase + goff, 8), GGRP)],
                        rbuf.at[pl.ds(0, GGRP)])
        pltpu.async_copy(t_hbm.at[sbuf], rows, sem).wait()

        def upd(j, _):
            r = rbuf[pl.ds(j, 16)][0] - base
            for f in range(8):
                sl = pl.ds(f * 16, 16)
                acc[r, sl] = jnp.maximum(acc[r, sl], rows[j, sl])
            return 0

        lax.fori_loop(0, GGRP, upd, 0)
        return 0

    lax.fori_loop(0, ngrp, grp_body, 0)

    pltpu.sync_copy(acc.at[pl.ds(0, RPW)], out_hbm.at[pl.ds(base, RPW)])


_segmax = functools.partial(
    pl.kernel,
    out_type=[
        jax.ShapeDtypeStruct((NPAD, D), jnp.float32),
        jax.ShapeDtypeStruct((NW * ECAP,), jnp.int32),
        jax.ShapeDtypeStruct((NW * ECAP,), jnp.int32),
    ],
    mesh=plsc.VectorSubcoreMesh(core_axis_name="c", subcore_axis_name="s"),
    scratch_types=[
        pltpu.VMEM((RPW + 8, D), jnp.float32),
        pltpu.VMEM((CH,), jnp.int32),
        pltpu.VMEM((CH,), jnp.int32),
        pltpu.VMEM((CH,), jnp.int32),
        pltpu.VMEM((2, 128), jnp.int32),
        pltpu.VMEM((GGRP,), jnp.int32),
        pltpu.VMEM((GGRP + 16,), jnp.int32),
        pltpu.VMEM((GGRP, D), jnp.float32),
        pltpu.SemaphoreType.DMA,
    ],
)(_segmax_body)


def kernel(node_features, senders, receivers, W_fs, b_fs, W_gn, b_gn, W_gin, b_gin):
    nb = N_NODES // BS
    transformed, base = pl.pallas_call(
        _mm1_body,
        grid=(nb,),
        in_specs=[
            pl.BlockSpec((BS, D), lambda i: (i, 0)),
            pl.BlockSpec((D, D), lambda i: (0, 0)),
            pl.BlockSpec((D,), lambda i: (0,)),
            pl.BlockSpec((D, D), lambda i: (0, 0)),
            pl.BlockSpec((D,), lambda i: (0,)),
        ],
        out_specs=[
            pl.BlockSpec((BS, D), lambda i: (i, 0)),
            pl.BlockSpec((BS, D), lambda i: (i, 0)),
        ],
        out_shape=[
            jax.ShapeDtypeStruct((N_NODES, D), jnp.float32),
            jax.ShapeDtypeStruct((N_NODES, D), jnp.float32),
        ],
    )(node_features, W_fs, b_fs, W_gn, b_gn)

    seg, _, _ = _segmax(transformed, senders, receivers)
    seg = seg[:N_NODES]

    nodes = pl.pallas_call(
        _mm2_body,
        grid=(nb,),
        in_specs=[
            pl.BlockSpec((BS, D), lambda i: (i, 0)),
            pl.BlockSpec((D, D), lambda i: (0, 0)),
            pl.BlockSpec((D,), lambda i: (0,)),
            pl.BlockSpec((BS, D), lambda i: (i, 0)),
        ],
        out_specs=pl.BlockSpec((BS, D), lambda i: (i, 0)),
        out_shape=jax.ShapeDtypeStruct((N_NODES, D), jnp.float32),
    )(seg, W_gin, b_gin, base)
    return nodes


# speedup vs baseline: 2.8436x; 2.8436x over previous
import functools

import jax
import jax.numpy as jnp
from jax import lax
from jax.experimental import pallas as pl
from jax.experimental.pallas import tpu as pltpu
from jax.experimental.pallas import tpu_sc as plsc

N_NODES = 10000
N_EDGES = 320000
D = 128
BS = 1000

NW = 32
RPW = 320
NPAD = NW * RPW
ECAP = N_EDGES + 8192
TRASHREG = 4096
CH = 2560
NV = CH // 16
NCH = N_EDGES // CH
NBLK = CH // 128
GGRP = 128


def _mm1_body(x_ref, wfs_ref, bfs_ref, wgn_ref, bgn_ref, t_ref, base_ref):
    x = x_ref[...]
    t_ref[...] = jnp.maximum(
        lax.dot_general(x, wfs_ref[...], (((1,), (1,)), ((), ()))) + bfs_ref[...], 0.0
    )
    base_ref[...] = lax.dot_general(x, wgn_ref[...], (((1,), (1,)), ((), ()))) + bgn_ref[...]


def _mm2_body(seg_ref, wgin_ref, bgin_ref, base_ref, out_ref):
    out_ref[...] = (
        lax.dot_general(seg_ref[...], wgin_ref[...], (((1,), (1,)), ((), ())))
        + bgin_ref[...]
        + base_ref[...]
    )


def _segmax_body(t_hbm, snd_hbm, rcv_hbm, out_hbm, wls_hbm, wlr_hbm,
                 acc, rv_buf, sv_buf, posbuf, padsrc, sbuf, rbuf, rows, sem):
    wid = lax.axis_index("s") * 2 + lax.axis_index("c")
    base = wid * RPW
    wbase = wid * ECAP
    trash_pos = wbase + ECAP - TRASHREG
    lane = lax.iota(jnp.int32, 16)
    dn = lax.GatherDimensionNumbers(offset_dims=(), collapsed_slice_dims=(0,),
                                    start_index_map=(0,))

    def dg(x, idx):
        return lax.gather(x, idx[:, None], dn, (1,),
                          mode=lax.GatherScatterMode.PROMISE_IN_BOUNDS)

    zero16f = jnp.zeros((16,), jnp.float32)
    zero16i = jnp.zeros((16,), jnp.int32)

    def zrow(i, _):
        for f in range(8):
            acc[i, pl.ds(f * 16, 16)] = zero16f
        return 0

    lax.fori_loop(0, RPW + 8, zrow, 0)

    def zpad(i, _):
        o = pl.multiple_of(i * 16, 16)
        padsrc[0, pl.ds(o, 16)] = zero16i
        padsrc[1, pl.ds(o, 16)] = zero16i + (base + RPW)
        return 0

    lax.fori_loop(0, 8, zpad, 0)

    def chunk_body(c, gcnt):
        off = pl.multiple_of(c * CH, 8)
        pltpu.sync_copy(rcv_hbm.at[pl.ds(off, CH)], rv_buf)
        pltpu.sync_copy(snd_hbm.at[pl.ds(off, CH)], sv_buf)

        def scan_body(i, gp):
            o = pl.multiple_of(i * 16, 16)
            rv = rv_buf[pl.ds(o, 16)]
            lowerm = jnp.minimum(jnp.maximum(rv - (base - 1), 0), 1)
            upperm = jnp.minimum(jnp.maximum((base + RPW) - rv, 0), 1)
            mi = lowerm * upperm
            x = mi
            for j in (1, 2, 4, 8):
                x = x + jnp.where(lane >= j, dg(x, jnp.maximum(lane - j, 0)), 0)
            a = x + (wbase + gp - 1)
            tv = lane + (trash_pos + o)
            posbuf[pl.ds(o, 16)] = mi * (a - tv) + tv
            return gp + x[15]

        gp2 = lax.fori_loop(0, NV, scan_body, gcnt)

        cps = []
        for k in range(NBLK):
            ko = k * 128
            cps.append(pltpu.async_copy(
                sv_buf.at[pl.ds(ko, 128)], wls_hbm.at[posbuf.at[pl.ds(ko, 128)]], sem))
            cps.append(pltpu.async_copy(
                rv_buf.at[pl.ds(ko, 128)], wlr_hbm.at[posbuf.at[pl.ds(ko, 128)]], sem))
        for cp in cps:
            cp.wait()
        return gp2

    gcnt = lax.fori_loop(0, NCH, chunk_body, jnp.int32(0))

    def padpos(i, _):
        o = pl.multiple_of(i * 16, 16)
        posbuf[pl.ds(o, 16)] = lane + (wbase + gcnt + o)
        return 0

    lax.fori_loop(0, 8, padpos, 0)
    pltpu.async_copy(padsrc.at[0, pl.ds(0, 128)],
                     wls_hbm.at[posbuf.at[pl.ds(0, 128)]], sem).wait()
    pltpu.async_copy(padsrc.at[1, pl.ds(0, 128)],
                     wlr_hbm.at[posbuf.at[pl.ds(0, 128)]], sem).wait()

    ngrp = (gcnt + (GGRP - 1)) // GGRP

    def grp_body(g, _):
        goff = pl.multiple_of(g * GGRP, GGRP)
        pltpu.sync_copy(wls_hbm.at[pl.ds(pl.multiple_of(wbase + goff, 8), GGRP)], sbuf)
        pltpu.sync_copy(wlr_hbm.at[pl.ds(pl.multiple_of(wbase + goff, 8), GGRP)],
                        rbuf.at[pl.ds(0, GGRP)])
        pltpu.async_copy(t_hbm.at[sbuf], rows, sem).wait()

        def upd(j, _):
            r = rbuf[pl.ds(j, 16)][0] - base
            for f in range(8):
                sl = pl.ds(f * 16, 16)
                acc[r, sl] = jnp.maximum(acc[r, sl], rows[j, sl])
            return 0

        lax.fori_loop(0, GGRP, upd, 0)
        return 0

    lax.fori_loop(0, ngrp, grp_body, 0)

    pltpu.sync_copy(acc.at[pl.ds(0, RPW)], out_hbm.at[pl.ds(base, RPW)])


_segmax = functools.partial(
    pl.kernel,
    out_type=[
        jax.ShapeDtypeStruct((NPAD, D), jnp.float32),
        jax.ShapeDtypeStruct((NW * ECAP,), jnp.int32),
        jax.ShapeDtypeStruct((NW * ECAP,), jnp.int32),
    ],
    mesh=plsc.VectorSubcoreMesh(core_axis_name="c", subcore_axis_name="s"),
    scratch_types=[
        pltpu.VMEM((RPW + 8, D), jnp.float32),
        pltpu.VMEM((CH,), jnp.int32),
        pltpu.VMEM((CH,), jnp.int32),
        pltpu.VMEM((CH,), jnp.int32),
        pltpu.VMEM((2, 128), jnp.int32),
        pltpu.VMEM((GGRP,), jnp.int32),
        pltpu.VMEM((GGRP + 16,), jnp.int32),
        pltpu.VMEM((GGRP, D), jnp.float32),
        pltpu.SemaphoreType.DMA,
    ],
)(_segmax_body)


def kernel(node_features, senders, receivers, W_fs, b_fs, W_gn, b_gn, W_gin, b_gin):
    nb = N_NODES // BS
    transformed, base = pl.pallas_call(
        _mm1_body,
        grid=(nb,),
        in_specs=[
            pl.BlockSpec((BS, D), lambda i: (i, 0)),
            pl.BlockSpec((D, D), lambda i: (0, 0)),
            pl.BlockSpec((D,), lambda i: (0,)),
            pl.BlockSpec((D, D), lambda i: (0, 0)),
            pl.BlockSpec((D,), lambda i: (0,)),
        ],
        out_specs=[
            pl.BlockSpec((BS, D), lambda i: (i, 0)),
            pl.BlockSpec((BS, D), lambda i: (i, 0)),
        ],
        out_shape=[
            jax.ShapeDtypeStruct((N_NODES, D), jnp.float32),
            jax.ShapeDtypeStruct((N_NODES, D), jnp.float32),
        ],
    )(node_features, W_fs, b_fs, W_gn, b_gn)

    seg, _, _ = _segmax(transformed, senders, receivers)
    seg = seg[:N_NODES]

    nodes = pl.pallas_call(
        _mm2_body,
        grid=(nb,),
        in_specs=[
            pl.BlockSpec((BS, D), lambda i: (i, 0)),
            pl.BlockSpec((D, D), lambda i: (0, 0)),
            pl.BlockSpec((D,), lambda i: (0,)),
            pl.BlockSpec((BS, D), lambda i: (i, 0)),
        ],
        out_specs=pl.BlockSpec((BS, D), lambda i: (i, 0)),
        out_shape=jax.ShapeDtypeStruct((N_NODES, D), jnp.float32),
    )(seg, W_gin, b_gin, base)
    return nodes


# speedup vs baseline: 61.5097x; 21.6313x over previous
import functools

import jax
import jax.numpy as jnp
from jax import lax
from jax.experimental import pallas as pl
from jax.experimental.pallas import tpu as pltpu
from jax.experimental.pallas import tpu_sc as plsc

N_NODES = 10000
N_EDGES = 320000
D = 128
BS = 1000

NW = 32
RPW = 320
NPAD = NW * RPW
CH = 1280
NV = CH // 16
NCH = N_EDGES // CH
PACK = 16384


def _mm1_body(x_ref, wfs_ref, bfs_ref, wgn_ref, bgn_ref, t_ref, base_ref):
    x = x_ref[...]
    t_ref[...] = jnp.maximum(
        lax.dot_general(x, wfs_ref[...], (((1,), (1,)), ((), ()))) + bfs_ref[...], 0.0
    )
    base_ref[...] = lax.dot_general(x, wgn_ref[...], (((1,), (1,)), ((), ()))) + bgn_ref[...]


def _mm2_body(seg_ref, wgin_ref, bgin_ref, base_ref, out_ref):
    out_ref[...] = (
        lax.dot_general(seg_ref[...], wgin_ref[...], (((1,), (1,)), ((), ())))
        + bgin_ref[...]
        + base_ref[...]
    )


def _segmax_body(t1_hbm, snd_hbm, rcv_hbm, out_hbm,
                 acc, rv_buf, sv_buf, rows, wl, sem):
    wid = lax.axis_index("s") * 2 + lax.axis_index("c")
    base = wid * RPW
    trash_q = (base + RPW) * PACK
    lane = lax.iota(jnp.int32, 16)
    dn = lax.GatherDimensionNumbers(offset_dims=(), collapsed_slice_dims=(0,),
                                    start_index_map=(0,))

    def dg(x, idx):
        return lax.gather(x, idx[:, None], dn, (1,),
                          mode=lax.GatherScatterMode.PROMISE_IN_BOUNDS)

    zero16f = jnp.zeros((16,), jnp.float32)

    def zrow(i, _):
        for f in range(8):
            acc[i, pl.ds(f * 16, 16)] = zero16f
        return 0

    lax.fori_loop(0, RPW + 8, zrow, 0)

    def chunk_body(c, _):
        off = pl.multiple_of(c * CH, 8)
        pltpu.sync_copy(rcv_hbm.at[pl.ds(off, CH)], rv_buf)
        pltpu.sync_copy(snd_hbm.at[pl.ds(off, CH)], sv_buf)

        def scan_body(i, cnt):
            o = pl.multiple_of(i * 16, 16)
            rv = rv_buf[pl.ds(o, 16)]
            lowerm = jnp.minimum(jnp.maximum(rv - (base - 1), 0), 1)
            upperm = jnp.minimum(jnp.maximum((base + RPW) - rv, 0), 1)
            mi = lowerm * upperm
            x = mi
            for j in (1, 2, 4, 8):
                x = x + jnp.where(lane >= j, dg(x, jnp.maximum(lane - j, 0)), 0)
            total = x[15]

            def do(cn):
                rv2 = rv_buf[pl.ds(o, 16)]
                sv2 = sv_buf[pl.ds(o, 16)]
                l2 = jnp.minimum(jnp.maximum(rv2 - (base - 1), 0), 1)
                u2 = jnp.minimum(jnp.maximum((base + RPW) - rv2, 0), 1)
                mi2 = l2 * u2
                pvv = mi2 * (sv2 + rv2 * PACK + 1)
                for l in range(16):
                    q = pvv[l]
                    wl[cn] = q - 1
                    cn = cn + jnp.minimum(q, 1)
                return cn

            return lax.cond(total > 0, do, lambda cn: cn, cnt)

        cnt = lax.fori_loop(0, NV, scan_body, jnp.int32(0))

        def padb(i, cn):
            wl[cn] = trash_q
            return cn + 1

        cnt = lax.fori_loop(0, (16 - cnt % 16) % 16, padb, cnt)

        def batch_body(b, _):
            jo = b * 16
            cps = []
            for k in range(16):
                q = wl[jo + k]
                s = q - (q // PACK) * PACK
                soff = pl.multiple_of(s * 128, 8)
                cps.append(pltpu.async_copy(
                    t1_hbm.at[pl.ds(soff, 128)], rows.at[k], sem))
            for cp in cps:
                cp.wait()
            for k in range(16):
                q = wl[jo + k]
                r = q // PACK - base
                for f in range(8):
                    sl = pl.ds(f * 16, 16)
                    acc[r, sl] = jnp.maximum(acc[r, sl], rows[k, sl])
            return 0

        lax.fori_loop(0, cnt // 16, batch_body, 0)
        return 0

    lax.fori_loop(0, NCH, chunk_body, 0)

    pltpu.sync_copy(acc.at[pl.ds(0, RPW)], out_hbm.at[pl.ds(base, RPW)])


_segmax = functools.partial(
    pl.kernel,
    out_type=jax.ShapeDtypeStruct((NPAD, D), jnp.float32),
    mesh=plsc.VectorSubcoreMesh(core_axis_name="c", subcore_axis_name="s"),
    scratch_types=[
        pltpu.VMEM((RPW + 8, D), jnp.float32),
        pltpu.VMEM((CH,), jnp.int32),
        pltpu.VMEM((CH,), jnp.int32),
        pltpu.VMEM((16, D), jnp.float32),
        pltpu.SMEM((CH + 16,), jnp.int32),
        pltpu.SemaphoreType.DMA,
    ],
)(_segmax_body)


def kernel(node_features, senders, receivers, W_fs, b_fs, W_gn, b_gn, W_gin, b_gin):
    nb = N_NODES // BS
    transformed, base = pl.pallas_call(
        _mm1_body,
        grid=(nb,),
        in_specs=[
            pl.BlockSpec((BS, D), lambda i: (i, 0)),
            pl.BlockSpec((D, D), lambda i: (0, 0)),
            pl.BlockSpec((D,), lambda i: (0,)),
            pl.BlockSpec((D, D), lambda i: (0, 0)),
            pl.BlockSpec((D,), lambda i: (0,)),
        ],
        out_specs=[
            pl.BlockSpec((BS, D), lambda i: (i, 0)),
            pl.BlockSpec((BS, D), lambda i: (i, 0)),
        ],
        out_shape=[
            jax.ShapeDtypeStruct((N_NODES, D), jnp.float32),
            jax.ShapeDtypeStruct((N_NODES, D), jnp.float32),
        ],
    )(node_features, W_fs, b_fs, W_gn, b_gn)

    seg = _segmax(transformed.reshape(N_NODES * D), senders, receivers)[:N_NODES]

    nodes = pl.pallas_call(
        _mm2_body,
        grid=(nb,),
        in_specs=[
            pl.BlockSpec((BS, D), lambda i: (i, 0)),
            pl.BlockSpec((D, D), lambda i: (0, 0)),
            pl.BlockSpec((D,), lambda i: (0,)),
            pl.BlockSpec((BS, D), lambda i: (i, 0)),
        ],
        out_specs=pl.BlockSpec((BS, D), lambda i: (i, 0)),
        out_shape=jax.ShapeDtypeStruct((N_NODES, D), jnp.float32),
    )(seg, W_gin, b_gin, base)
    return nodes


# speedup vs baseline: 87.8423x; 1.4281x over previous
import functools

import jax
import jax.numpy as jnp
from jax import lax
from jax.experimental import pallas as pl
from jax.experimental.pallas import tpu as pltpu
from jax.experimental.pallas import tpu_sc as plsc

N_NODES = 10000
N_EDGES = 320000
D = 128
BS = 1000

NW = 32
RPW = 320
NPAD = NW * RPW
CH = 1280
NV = CH // 16
NCH = N_EDGES // CH
PACK = 16384


def _mm1_body(x_ref, wfs_ref, bfs_ref, wgn_ref, bgn_ref, t_ref, base_ref):
    x = x_ref[...]
    t_ref[...] = jnp.maximum(
        lax.dot_general(x, wfs_ref[...], (((1,), (1,)), ((), ()))) + bfs_ref[...], 0.0
    )
    base_ref[...] = lax.dot_general(x, wgn_ref[...], (((1,), (1,)), ((), ()))) + bgn_ref[...]


def _mm2_body(seg_ref, wgin_ref, bgin_ref, base_ref, out_ref):
    out_ref[...] = (
        lax.dot_general(seg_ref[...], wgin_ref[...], (((1,), (1,)), ((), ())))
        + bgin_ref[...]
        + base_ref[...]
    )


def _segmax_body(t_hbm, snd_hbm, rcv_hbm, out_hbm,
                 acc, rv_buf, sv_buf, sidx, rows, wl, sem):
    wid = lax.axis_index("s") * 2 + lax.axis_index("c")
    base = wid * RPW
    trash_q = (base + RPW) * PACK
    lane = lax.iota(jnp.int32, 16)
    dn = lax.GatherDimensionNumbers(offset_dims=(), collapsed_slice_dims=(0,),
                                    start_index_map=(0,))

    def dg(x, idx):
        return lax.gather(x, idx[:, None], dn, (1,),
                          mode=lax.GatherScatterMode.PROMISE_IN_BOUNDS)

    zero16f = jnp.zeros((16,), jnp.float32)

    def zrow(i, _):
        for f in range(8):
            acc[i, pl.ds(f * 16, 16)] = zero16f
        return 0

    lax.fori_loop(0, RPW + 8, zrow, 0)

    def process_group(jo):
        for v in range(8):
            svec = lane * 0
            for k in range(16):
                q = wl[jo + v * 16 + k]
                s = q - (q // PACK) * PACK
                svec = jnp.where(lane == k, s, svec)
            sidx[pl.ds(v * 16, 16)] = svec
        pltpu.async_copy(t_hbm.at[sidx], rows, sem).wait()

        def upd(j, _):
            q = wl[jo + j]
            r = q // PACK - base
            for f in range(8):
                sl = pl.ds(f * 16, 16)
                acc[r, sl] = jnp.maximum(acc[r, sl], rows[j, sl])
            return 0

        lax.fori_loop(0, 128, upd, 0)

    def chunk_body(c, cin):
        off = pl.multiple_of(c * CH, 8)
        pltpu.sync_copy(rcv_hbm.at[pl.ds(off, CH)], rv_buf)
        pltpu.sync_copy(snd_hbm.at[pl.ds(off, CH)], sv_buf)

        def scan_body(i, cnt):
            o = pl.multiple_of(i * 16, 16)
            rv = rv_buf[pl.ds(o, 16)]
            lowerm = jnp.minimum(jnp.maximum(rv - (base - 1), 0), 1)
            upperm = jnp.minimum(jnp.maximum((base + RPW) - rv, 0), 1)
            mi = lowerm * upperm
            x = mi
            for j in (1, 2, 4, 8):
                x = x + jnp.where(lane >= j, dg(x, jnp.maximum(lane - j, 0)), 0)
            total = x[15]

            def do(cn):
                rv2 = rv_buf[pl.ds(o, 16)]
                sv2 = sv_buf[pl.ds(o, 16)]
                l2 = jnp.minimum(jnp.maximum(rv2 - (base - 1), 0), 1)
                u2 = jnp.minimum(jnp.maximum((base + RPW) - rv2, 0), 1)
                mi2 = l2 * u2
                pvv = mi2 * (sv2 + rv2 * PACK + 1)
                for l in range(16):
                    q = pvv[l]
                    wl[cn] = q - 1
                    cn = cn + jnp.minimum(q, 1)
                return cn

            return lax.cond(total > 0, do, lambda cn: cn, cnt)

        cnt = lax.fori_loop(0, NV, scan_body, cin)

        ngrp = cnt // 128

        def grp_body(g, _):
            process_group(g * 128)
            return 0

        lax.fori_loop(0, ngrp, grp_body, 0)

        rem = cnt - ngrp * 128

        def mv(i, _):
            wl[i] = wl[ngrp * 128 + i]
            return 0

        lax.fori_loop(0, rem, mv, 0)
        return rem

    cnt = lax.fori_loop(0, NCH, chunk_body, jnp.int32(0))

    def padb(i, cn):
        wl[cn] = trash_q
        return cn + 1

    cnt = lax.fori_loop(0, (128 - cnt % 128) % 128, padb, cnt)

    def last_body(g, _):
        process_group(g * 128)
        return 0

    lax.fori_loop(0, cnt // 128, last_body, 0)

    pltpu.sync_copy(acc.at[pl.ds(0, RPW)], out_hbm.at[pl.ds(base, RPW)])


_segmax = functools.partial(
    pl.kernel,
    out_type=jax.ShapeDtypeStruct((NPAD, D), jnp.float32),
    mesh=plsc.VectorSubcoreMesh(core_axis_name="c", subcore_axis_name="s"),
    scratch_types=[
        pltpu.VMEM((RPW + 8, D), jnp.float32),
        pltpu.VMEM((CH,), jnp.int32),
        pltpu.VMEM((CH,), jnp.int32),
        pltpu.VMEM((128,), jnp.int32),
        pltpu.VMEM((128, D), jnp.float32),
        pltpu.SMEM((CH + 256,), jnp.int32),
        pltpu.SemaphoreType.DMA,
    ],
)(_segmax_body)


def kernel(node_features, senders, receivers, W_fs, b_fs, W_gn, b_gn, W_gin, b_gin):
    nb = N_NODES // BS
    transformed, base = pl.pallas_call(
        _mm1_body,
        grid=(nb,),
        in_specs=[
            pl.BlockSpec((BS, D), lambda i: (i, 0)),
            pl.BlockSpec((D, D), lambda i: (0, 0)),
            pl.BlockSpec((D,), lambda i: (0,)),
            pl.BlockSpec((D, D), lambda i: (0, 0)),
            pl.BlockSpec((D,), lambda i: (0,)),
        ],
        out_specs=[
            pl.BlockSpec((BS, D), lambda i: (i, 0)),
            pl.BlockSpec((BS, D), lambda i: (i, 0)),
        ],
        out_shape=[
            jax.ShapeDtypeStruct((N_NODES, D), jnp.float32),
            jax.ShapeDtypeStruct((N_NODES, D), jnp.float32),
        ],
    )(node_features, W_fs, b_fs, W_gn, b_gn)

    seg = _segmax(transformed, senders, receivers)[:N_NODES]

    nodes = pl.pallas_call(
        _mm2_body,
        grid=(nb,),
        in_specs=[
            pl.BlockSpec((BS, D), lambda i: (i, 0)),
            pl.BlockSpec((D, D), lambda i: (0, 0)),
            pl.BlockSpec((D,), lambda i: (0,)),
            pl.BlockSpec((BS, D), lambda i: (i, 0)),
        ],
        out_specs=pl.BlockSpec((BS, D), lambda i: (i, 0)),
        out_shape=jax.ShapeDtypeStruct((N_NODES, D), jnp.float32),
    )(seg, W_gin, b_gin, base)
    return nodes


# speedup vs baseline: 103.3087x; 1.1761x over previous
import functools

import jax
import jax.numpy as jnp
from jax import lax
from jax.experimental import pallas as pl
from jax.experimental.pallas import tpu as pltpu
from jax.experimental.pallas import tpu_sc as plsc

N_NODES = 10000
N_EDGES = 320000
D = 128
BS = 1000

NW = 32
RPW = 320
NPAD = NW * RPW
CH = 1280
NV = CH // 16
NCH = N_EDGES // CH
PACK = 16384


def _mm1_body(x_ref, wfs_ref, bfs_ref, wgn_ref, bgn_ref, t_ref, base_ref):
    x = x_ref[...]
    t_ref[...] = jnp.maximum(
        lax.dot_general(x, wfs_ref[...], (((1,), (1,)), ((), ()))) + bfs_ref[...], 0.0
    )
    base_ref[...] = lax.dot_general(x, wgn_ref[...], (((1,), (1,)), ((), ()))) + bgn_ref[...]


def _mm2_body(seg_ref, wgin_ref, bgin_ref, base_ref, out_ref):
    out_ref[...] = (
        lax.dot_general(seg_ref[...], wgin_ref[...], (((1,), (1,)), ((), ())))
        + bgin_ref[...]
        + base_ref[...]
    )


def _segmax_body(t_hbm, snd_hbm, rcv_hbm, out_hbm,
                 acc, rv_buf, sv_buf, rv_buf2, sv_buf2, sidx, rows, wl, sem, sem2):
    wid = lax.axis_index("s") * 2 + lax.axis_index("c")
    base = wid * RPW
    trash_q = (base + RPW) * PACK
    lane = lax.iota(jnp.int32, 16)
    dn = lax.GatherDimensionNumbers(offset_dims=(), collapsed_slice_dims=(0,),
                                    start_index_map=(0,))

    def dg(x, idx):
        return lax.gather(x, idx[:, None], dn, (1,),
                          mode=lax.GatherScatterMode.PROMISE_IN_BOUNDS)

    zero16f = jnp.zeros((16,), jnp.float32)

    def zrow(i, _):
        for f in range(8):
            acc[i, pl.ds(f * 16, 16)] = zero16f
        return 0

    lax.fori_loop(0, RPW + 8, zrow, 0)

    def process_group(jo):
        for v in range(8):
            svec = lane * 0
            for k in range(16):
                q = wl[jo + v * 16 + k]
                s = q - (q // PACK) * PACK
                svec = jnp.where(lane == k, s, svec)
            sidx[pl.ds(v * 16, 16)] = svec
        pltpu.async_copy(t_hbm.at[sidx], rows, sem).wait()

        def upd(j, _):
            q = wl[jo + j]
            r = q // PACK - base
            for f in range(8):
                sl = pl.ds(f * 16, 16)
                acc[r, sl] = jnp.maximum(acc[r, sl], rows[j, sl])
            return 0

        lax.fori_loop(0, 128, upd, 0)

    def scan_chunk(rvb, svb, cin):
        def scan_body(i, cnt):
            o = pl.multiple_of(i * 16, 16)
            rv = rvb[pl.ds(o, 16)]
            lowerm = jnp.minimum(jnp.maximum(rv - (base - 1), 0), 1)
            upperm = jnp.minimum(jnp.maximum((base + RPW) - rv, 0), 1)
            mi = lowerm * upperm
            x = mi
            for j in (1, 2, 4, 8):
                x = x + jnp.where(lane >= j, dg(x, jnp.maximum(lane - j, 0)), 0)
            total = x[15]

            def do(cn):
                rv2 = rvb[pl.ds(o, 16)]
                sv2 = svb[pl.ds(o, 16)]
                l2 = jnp.minimum(jnp.maximum(rv2 - (base - 1), 0), 1)
                u2 = jnp.minimum(jnp.maximum((base + RPW) - rv2, 0), 1)
                mi2 = l2 * u2
                pvv = mi2 * (sv2 + rv2 * PACK + 1)
                for l in range(16):
                    q = pvv[l]
                    wl[cn] = q - 1
                    cn = cn + jnp.minimum(q, 1)
                return cn

            return lax.cond(total > 0, do, lambda cn: cn, cnt)

        cnt = lax.fori_loop(0, NV, scan_body, cin)

        ngrp = cnt // 128

        def grp_body(g, _):
            process_group(g * 128)
            return 0

        lax.fori_loop(0, ngrp, grp_body, 0)

        rem = cnt - ngrp * 128

        def mv(i, _):
            wl[i] = wl[ngrp * 128 + i]
            return 0

        lax.fori_loop(0, rem, mv, 0)
        return rem

    def _issue(c, rvb, svb):
        off = pl.multiple_of((c % NCH) * CH, 8)
        pltpu.async_copy(rcv_hbm.at[pl.ds(off, CH)], rvb, sem2)
        pltpu.async_copy(snd_hbm.at[pl.ds(off, CH)], svb, sem2)

    def _drain(rvb, svb):
        pltpu.make_async_copy(rcv_hbm.at[pl.ds(0, CH)], rvb, sem2).wait()
        pltpu.make_async_copy(snd_hbm.at[pl.ds(0, CH)], svb, sem2).wait()

    _issue(0, rv_buf, sv_buf)

    def pair_body(p, cin):
        c0 = p * 2
        _issue(c0 + 1, rv_buf2, sv_buf2)
        _drain(rv_buf, sv_buf)
        cnt = scan_chunk(rv_buf, sv_buf, cin)
        _issue(c0 + 2, rv_buf, sv_buf)
        _drain(rv_buf2, sv_buf2)
        cnt = scan_chunk(rv_buf2, sv_buf2, cnt)
        return cnt

    cnt = lax.fori_loop(0, NCH // 2, pair_body, jnp.int32(0))
    _drain(rv_buf, sv_buf)

    def padb(i, cn):
        wl[cn] = trash_q
        return cn + 1

    cnt = lax.fori_loop(0, (128 - cnt % 128) % 128, padb, cnt)

    def last_body(g, _):
        process_group(g * 128)
        return 0

    lax.fori_loop(0, cnt // 128, last_body, 0)

    pltpu.sync_copy(acc.at[pl.ds(0, RPW)], out_hbm.at[pl.ds(base, RPW)])


_segmax = functools.partial(
    pl.kernel,
    out_type=jax.ShapeDtypeStruct((NPAD, D), jnp.float32),
    mesh=plsc.VectorSubcoreMesh(core_axis_name="c", subcore_axis_name="s"),
    scratch_types=[
        pltpu.VMEM((RPW + 8, D), jnp.float32),
        pltpu.VMEM((CH,), jnp.int32),
        pltpu.VMEM((CH,), jnp.int32),
        pltpu.VMEM((CH,), jnp.int32),
        pltpu.VMEM((CH,), jnp.int32),
        pltpu.VMEM((128,), jnp.int32),
        pltpu.VMEM((128, D), jnp.float32),
        pltpu.SMEM((CH + 256,), jnp.int32),
        pltpu.SemaphoreType.DMA,
        pltpu.SemaphoreType.DMA,
    ],
)(_segmax_body)


def kernel(node_features, senders, receivers, W_fs, b_fs, W_gn, b_gn, W_gin, b_gin):
    nb = N_NODES // BS
    transformed, base = pl.pallas_call(
        _mm1_body,
        grid=(nb,),
        in_specs=[
            pl.BlockSpec((BS, D), lambda i: (i, 0)),
            pl.BlockSpec((D, D), lambda i: (0, 0)),
            pl.BlockSpec((D,), lambda i: (0,)),
            pl.BlockSpec((D, D), lambda i: (0, 0)),
            pl.BlockSpec((D,), lambda i: (0,)),
        ],
        out_specs=[
            pl.BlockSpec((BS, D), lambda i: (i, 0)),
            pl.BlockSpec((BS, D), lambda i: (i, 0)),
        ],
        out_shape=[
            jax.ShapeDtypeStruct((N_NODES, D), jnp.float32),
            jax.ShapeDtypeStruct((N_NODES, D), jnp.float32),
        ],
    )(node_features, W_fs, b_fs, W_gn, b_gn)

    seg = _segmax(transformed, senders, receivers)[:N_NODES]

    nodes = pl.pallas_call(
        _mm2_body,
        grid=(nb,),
        in_specs=[
            pl.BlockSpec((BS, D), lambda i: (i, 0)),
            pl.BlockSpec((D, D), lambda i: (0, 0)),
            pl.BlockSpec((D,), lambda i: (0,)),
            pl.BlockSpec((BS, D), lambda i: (i, 0)),
        ],
        out_specs=pl.BlockSpec((BS, D), lambda i: (i, 0)),
        out_shape=jax.ShapeDtypeStruct((N_NODES, D), jnp.float32),
    )(seg, W_gin, b_gin, base)
    return nodes


# speedup vs baseline: 103.9243x; 1.0060x over previous
import functools

import jax
import jax.numpy as jnp
from jax import lax
from jax.experimental import pallas as pl
from jax.experimental.pallas import tpu as pltpu
from jax.experimental.pallas import tpu_sc as plsc

N_NODES = 10000
N_EDGES = 320000
D = 128
BS = 1000

NW = 32
RPW = 320
NPAD = NW * RPW
CH = 1280
NV = CH // 16
NCH = N_EDGES // CH
PACK = 16384
DUMPOFF = 250


def _mm1_body(x_ref, wfs_ref, bfs_ref, wgn_ref, bgn_ref, t_ref, base_ref):
    x = x_ref[...]
    t_ref[...] = jnp.maximum(
        lax.dot_general(x, wfs_ref[...], (((1,), (1,)), ((), ()))) + bfs_ref[...], 0.0
    )
    base_ref[...] = lax.dot_general(x, wgn_ref[...], (((1,), (1,)), ((), ()))) + bgn_ref[...]


def _mm2_body(seg_ref, wgin_ref, bgin_ref, base_ref, out_ref):
    out_ref[...] = (
        lax.dot_general(seg_ref[...], wgin_ref[...], (((1,), (1,)), ((), ())))
        + bgin_ref[...]
        + base_ref[...]
    )


def _segmax_body(t_hbm, snd_hbm, rcv_hbm, out_hbm,
                 acc, rv_buf, sv_buf, rv_buf2, sv_buf2, sidx, rows, wl, sem, sem2):
    wid = lax.axis_index("s") * 2 + lax.axis_index("c")
    base = wid * RPW
    trash_q = (base + RPW) * PACK
    lane = lax.iota(jnp.int32, 16)
    dn = lax.GatherDimensionNumbers(offset_dims=(), collapsed_slice_dims=(0,),
                                    start_index_map=(0,))

    def dg(x, idx):
        return lax.gather(x, idx[:, None], dn, (1,),
                          mode=lax.GatherScatterMode.PROMISE_IN_BOUNDS)

    zero16f = jnp.zeros((16,), jnp.float32)

    def zrow(i, _):
        for f in range(8):
            acc[i, pl.ds(f * 16, 16)] = zero16f
        return 0

    lax.fori_loop(0, RPW + 8, zrow, 0)

    def process_group(jo):
        def build_half(h):
            for v in range(4 * h, 4 * h + 4):
                svec = lane * 0
                for k in range(16):
                    q = wl[jo + v * 16 + k]
                    s = q - (q // PACK) * PACK
                    svec = jnp.where(lane == k, s, svec)
                sidx[pl.ds(v * 16, 16)] = svec

        def upd_half(h):
            def upd(j, _):
                q = wl[jo + h * 64 + j]
                r = q // PACK - base
                for f in range(8):
                    sl = pl.ds(f * 16, 16)
                    acc[r, sl] = jnp.maximum(
                        acc[r, sl], rows[h * 64 + j, sl])
                return 0

            lax.fori_loop(0, 64, upd, 0)

        build_half(0)
        ha = pltpu.async_copy(t_hbm.at[sidx.at[pl.ds(0, 64)]],
                              rows.at[pl.ds(0, 64)], sem)
        build_half(1)
        hb = pltpu.async_copy(t_hbm.at[sidx.at[pl.ds(64, 64)]],
                              rows.at[pl.ds(64, 64)], sem)
        ha.wait()
        upd_half(0)
        hb.wait()
        upd_half(1)

    def scan_chunk(rvb, svb, cin):
        def scan_body(i, cnt):
            o = pl.multiple_of(i * 16, 16)
            rv = rvb[pl.ds(o, 16)]
            lowerm = jnp.minimum(jnp.maximum(rv - (base - 1), 0), 1)
            upperm = jnp.minimum(jnp.maximum((base + RPW) - rv, 0), 1)
            mi = lowerm * upperm
            x = mi
            for j in (1, 2, 4, 8):
                x = x + jnp.where(lane >= j, dg(x, jnp.maximum(lane - j, 0)), 0)
            total = x[15]

            def do(cn):
                rv2 = rvb[pl.ds(o, 16)]
                sv2 = svb[pl.ds(o, 16)]
                l2 = jnp.minimum(jnp.maximum(rv2 - (base - 1), 0), 1)
                u2 = jnp.minimum(jnp.maximum((base + RPW) - rv2, 0), 1)
                mi2 = l2 * u2
                x2 = mi2
                for j in (1, 2, 4, 8):
                    x2 = x2 + jnp.where(lane >= j, dg(x2, jnp.maximum(lane - j, 0)), 0)
                pvv = mi2 * (sv2 + rv2 * PACK + 1) - 1
                posn = mi2 * (x2 - 1 - DUMPOFF) + DUMPOFF
                for l in range(16):
                    wl[cn + posn[l]] = pvv[l]
                return cn + total

            return lax.cond(total > 0, do, lambda cn: cn, cnt)

        cnt = lax.fori_loop(0, NV, scan_body, cin)

        ngrp = cnt // 128

        def grp_body(g, _):
            process_group(g * 128)
            return 0

        lax.fori_loop(0, ngrp, grp_body, 0)

        rem = cnt - ngrp * 128

        def mv(i, _):
            wl[i] = wl[ngrp * 128 + i]
            return 0

        lax.fori_loop(0, rem, mv, 0)
        return rem

    def _issue(c, rvb, svb):
        off = pl.multiple_of((c % NCH) * CH, 8)
        pltpu.async_copy(rcv_hbm.at[pl.ds(off, CH)], rvb, sem2)
        pltpu.async_copy(snd_hbm.at[pl.ds(off, CH)], svb, sem2)

    def _drain(rvb, svb):
        pltpu.make_async_copy(rcv_hbm.at[pl.ds(0, CH)], rvb, sem2).wait()
        pltpu.make_async_copy(snd_hbm.at[pl.ds(0, CH)], svb, sem2).wait()

    _issue(0, rv_buf, sv_buf)

    def pair_body(p, cin):
        c0 = p * 2
        _issue(c0 + 1, rv_buf2, sv_buf2)
        _drain(rv_buf, sv_buf)
        cnt = scan_chunk(rv_buf, sv_buf, cin)
        _issue(c0 + 2, rv_buf, sv_buf)
        _drain(rv_buf2, sv_buf2)
        cnt = scan_chunk(rv_buf2, sv_buf2, cnt)
        return cnt

    cnt = lax.fori_loop(0, NCH // 2, pair_body, jnp.int32(0))
    _drain(rv_buf, sv_buf)

    def padb(i, cn):
        wl[cn] = trash_q
        return cn + 1

    cnt = lax.fori_loop(0, (128 - cnt % 128) % 128, padb, cnt)

    def last_body(g, _):
        process_group(g * 128)
        return 0

    lax.fori_loop(0, cnt // 128, last_body, 0)

    pltpu.sync_copy(acc.at[pl.ds(0, RPW)], out_hbm.at[pl.ds(base, RPW)])


_segmax = functools.partial(
    pl.kernel,
    out_type=jax.ShapeDtypeStruct((NPAD, D), jnp.float32),
    mesh=plsc.VectorSubcoreMesh(core_axis_name="c", subcore_axis_name="s"),
    scratch_types=[
        pltpu.VMEM((RPW + 8, D), jnp.float32),
        pltpu.VMEM((CH,), jnp.int32),
        pltpu.VMEM((CH,), jnp.int32),
        pltpu.VMEM((CH,), jnp.int32),
        pltpu.VMEM((CH,), jnp.int32),
        pltpu.VMEM((128,), jnp.int32),
        pltpu.VMEM((128, D), jnp.float32),
        pltpu.SMEM((CH + 384,), jnp.int32),
        pltpu.SemaphoreType.DMA,
        pltpu.SemaphoreType.DMA,
    ],
)(_segmax_body)


def kernel(node_features, senders, receivers, W_fs, b_fs, W_gn, b_gn, W_gin, b_gin):
    nb = N_NODES // BS
    transformed, base = pl.pallas_call(
        _mm1_body,
        grid=(nb,),
        in_specs=[
            pl.BlockSpec((BS, D), lambda i: (i, 0)),
            pl.BlockSpec((D, D), lambda i: (0, 0)),
            pl.BlockSpec((D,), lambda i: (0,)),
            pl.BlockSpec((D, D), lambda i: (0, 0)),
            pl.BlockSpec((D,), lambda i: (0,)),
        ],
        out_specs=[
            pl.BlockSpec((BS, D), lambda i: (i, 0)),
            pl.BlockSpec((BS, D), lambda i: (i, 0)),
        ],
        out_shape=[
            jax.ShapeDtypeStruct((N_NODES, D), jnp.float32),
            jax.ShapeDtypeStruct((N_NODES, D), jnp.float32),
        ],
    )(node_features, W_fs, b_fs, W_gn, b_gn)

    seg = _segmax(transformed, senders, receivers)[:N_NODES]

    nodes = pl.pallas_call(
        _mm2_body,
        grid=(nb,),
        in_specs=[
            pl.BlockSpec((BS, D), lambda i: (i, 0)),
            pl.BlockSpec((D, D), lambda i: (0, 0)),
            pl.BlockSpec((D,), lambda i: (0,)),
            pl.BlockSpec((BS, D), lambda i: (i, 0)),
        ],
        out_specs=pl.BlockSpec((BS, D), lambda i: (i, 0)),
        out_shape=jax.ShapeDtypeStruct((N_NODES, D), jnp.float32),
    )(seg, W_gin, b_gin, base)
    return nodes
